# baseline (speedup 1.0000x reference)
import functools

import jax
import jax.numpy as jnp
from jax.experimental import pallas as pl
from jax.experimental.pallas import tpu as pltpu
from jax.experimental.pallas import tpu_sc as plsc

_DN = (((1,), (1,)), ((), ()))


def _sc_scan(keynsp, nidxsp, kml, kmidx, *, B, S):
    J = S // 16
    JB = J // 16
    mesh = plsc.VectorSubcoreMesh(core_axis_name="c", subcore_axis_name="s")

    @functools.partial(
        pl.kernel,
        mesh=mesh,
        out_type=jax.ShapeDtypeStruct((B, 16, J), jnp.float32),
        scratch_types=[
            pltpu.VMEM((S * 16,), jnp.int32),
            pltpu.VMEM((S * 16,), jnp.int32),
            pltpu.VMEM((J,), jnp.int32),
            pltpu.VMEM((J,), jnp.int32),
            pltpu.VMEM((J,), jnp.float32),
            pltpu.VMEM((16,), jnp.int32),
        ],
    )
    def scan(keynsp_hbm, nidxsp_hbm, kml_hbm, kmidx_hbm, cnt_hbm,
             keynsp_v, nidxsp_v, kml_v, kmidx_v, out_v, acc_v):
        c = jax.lax.axis_index("c")
        s = jax.lax.axis_index("s")
        pltpu.sync_copy(keynsp_hbm.at[pl.ds(c * S * 16, S * 16)], keynsp_v)
        pltpu.sync_copy(nidxsp_hbm, nidxsp_v)
        pltpu.sync_copy(kml_hbm.at[pl.ds((c * 16 + s) * J, J)], kml_v)
        pltpu.sync_copy(kmidx_hbm.at[pl.ds(s * J, J)], kmidx_v)
        one = jnp.ones((16,), jnp.int32)
        zero = jnp.zeros((16,), jnp.int32)
        for jblk in range(JB):
            km_v = kml_v[pl.ds(jblk * 16, 16)]
            mi_v = kmidx_v[pl.ds(jblk * 16, 16)]
            def _bulk(n, acc, km_v=km_v):
                knsp = keynsp_v[pl.ds(n * 16, 16)]
                return acc + jnp.where(km_v == knsp, one, zero)

            def _edge(n, acc, km_v=km_v, mi_v=mi_v):
                knsp = keynsp_v[pl.ds(n * 16, 16)]
                nsp = nidxsp_v[pl.ds(n * 16, 16)]
                hit = (km_v == knsp) & (nsp < mi_v)
                return acc + jnp.where(hit, one, zero)

            acc = pl.loop(0, 256 * jblk, init_carry=zero, unroll=16)(_bulk)
            acc = pl.loop(256 * jblk, 256 * (jblk + 1), init_carry=acc,
                          unroll=16)(_edge)
            out_v[pl.ds(jblk * 16, 16)] = acc.astype(jnp.float32)
        pltpu.sync_copy(out_v, cnt_hbm.at[c, s])

    return scan(keynsp, nidxsp, kml, kmidx)


def _tc_body(keym_ref, keyn_ref, cnt_ref, xrow_ref, w0_ref, w1_ref, bias_ref,
             x_hbm_ref, y_ref, xblk_ref, sem, *, bm, bn, nblks, mpb):
    r = pl.program_id(0)
    b = r // mpb
    mi = jax.lax.rem(r, mpb)

    y_ref[...] = jax.lax.dot_general(
        xrow_ref[...], w1_ref[...], _DN, preferred_element_type=jnp.float32
    ) + bias_ref[...]

    cnt = cnt_ref[...]

    @pl.when(jnp.sum(cnt) > 0)
    def _correct():
        scale = jnp.where(cnt > 0, 1.0 / jnp.where(cnt > 0, cnt, 1.0), 0.0)
        keym = keym_ref[...]
        m_glob = mi * bm + jax.lax.broadcasted_iota(jnp.int32, (bm, 1), 0)
        for nb in range(nblks):
            keyn = keyn_ref[0, :, nb * bn:(nb + 1) * bn]
            n_glob = nb * bn + jax.lax.broadcasted_iota(
                jnp.int32, (bm, bn), 1)
            maskf = ((keym == keyn) & (n_glob < m_glob)).astype(jnp.float32)

            @pl.when(jnp.sum(maskf) > 0)
            def _acc(maskf=maskf, nb=nb):
                cp = pltpu.make_async_copy(
                    x_hbm_ref.at[b, pl.ds(nb * bn, bn), :], xblk_ref, sem)
                cp.start()
                cp.wait()
                z0b = jax.lax.dot_general(
                    xblk_ref[...], w0_ref[...], _DN,
                    preferred_element_type=jnp.float32)
                y_ref[...] += jnp.dot(maskf * scale, z0b,
                                      preferred_element_type=jnp.float32)


def kernel(x, input_ids, W0, b0, W1, b1):
    B, S, D = x.shape
    bm, bn = 512, 256
    mpb = S // bm
    nblks = S // bn
    R = B * S
    J = S // 16

    ids = input_ids.astype(jnp.int32)
    key = ids[:, :-1] * 1000 + ids[:, 1:]
    keyM = jnp.concatenate(
        [jnp.full((B, 1), -1, jnp.int32), key], axis=1)
    keyN = jnp.concatenate(
        [jnp.full((B, 2), -2, jnp.int32), key[:, :-1]], axis=1)
    bias = (b0 + b1).reshape(1, D)

    kml = keyM.reshape(B, J, 16).transpose(0, 2, 1).reshape(B * 16 * J)
    kmidx = (jnp.arange(16, dtype=jnp.int32)[:, None]
             + 16 * jnp.arange(J, dtype=jnp.int32)[None, :]).reshape(16 * J)
    keynsp = jnp.broadcast_to(keyN[:, :, None], (B, S, 16)).reshape(B * S * 16)
    nidxsp = jnp.broadcast_to(
        jnp.arange(S, dtype=jnp.int32)[:, None], (S, 16)).reshape(S * 16)
    cnt_perm = _sc_scan(keynsp, nidxsp, kml, kmidx, B=B, S=S)
    cnt = cnt_perm.transpose(0, 2, 1).reshape(R, 1)

    y = pl.pallas_call(
        functools.partial(_tc_body, bm=bm, bn=bn, nblks=nblks, mpb=mpb),
        grid=(R // bm,),
        in_specs=[
            pl.BlockSpec((bm, 1), lambda r: (r, 0)),
            pl.BlockSpec((1, 1, S), lambda r: (r // mpb, 0, 0)),
            pl.BlockSpec((bm, 1), lambda r: (r, 0)),
            pl.BlockSpec((bm, D), lambda r: (r, 0)),
            pl.BlockSpec((D, D), lambda r: (0, 0)),
            pl.BlockSpec((D, D), lambda r: (0, 0)),
            pl.BlockSpec((1, D), lambda r: (0, 0)),
            pl.BlockSpec(memory_space=pltpu.MemorySpace.HBM),
        ],
        out_specs=pl.BlockSpec((bm, D), lambda r: (r, 0)),
        out_shape=jax.ShapeDtypeStruct((R, D), jnp.float32),
        scratch_shapes=[
            pltpu.VMEM((bn, D), jnp.float32),
            pltpu.SemaphoreType.DMA,
        ],
        compiler_params=pltpu.CompilerParams(
            dimension_semantics=("arbitrary",),
            vmem_limit_bytes=62 * 1024 * 1024),
    )(keyM.reshape(R, 1), keyN[:, None, :], cnt, x.reshape(R, D), W0, W1,
      bias, x)
    return y.reshape(B, S, D)


# speedup vs baseline: 1.5523x; 1.5523x over previous
import functools

import jax
import jax.numpy as jnp
from jax.experimental import pallas as pl
from jax.experimental.pallas import tpu as pltpu

_DN = (((1,), (1,)), ((), ()))


def _body(keym_ref, keyn_ref, xrow_ref, w0_ref, w1_ref, bias_ref, x_hbm_ref,
          y_ref, xblk_ref, sem, *, bm, bn, nblks, mpb):
    r = pl.program_id(0)
    b = r // mpb
    mi = jax.lax.rem(r, mpb)

    y_ref[...] = jax.lax.dot_general(
        xrow_ref[...], w1_ref[...], _DN, preferred_element_type=jnp.float32
    ) + bias_ref[...]

    keym = keym_ref[...]
    m_glob = mi * bm + jax.lax.broadcasted_iota(jnp.int32, (bm, 1), 0)

    def mask_block(nb):
        keyn = keyn_ref[0, :, nb * bn:(nb + 1) * bn]
        n_glob = nb * bn + jax.lax.broadcasted_iota(jnp.int32, (bm, bn), 1)
        return ((keym == keyn) & (n_glob < m_glob)).astype(jnp.float32)

    rowsums = [jnp.sum(mask_block(nb), axis=1, keepdims=True)
               for nb in range(nblks)]
    cnt = sum(rowsums)

    @pl.when(jnp.sum(cnt) > 0)
    def _correct():
        scale = jnp.where(cnt > 0, 1.0 / jnp.where(cnt > 0, cnt, 1.0), 0.0)
        for nb in range(nblks):

            @pl.when(jnp.sum(rowsums[nb]) > 0)
            def _acc(nb=nb):
                cp = pltpu.make_async_copy(
                    x_hbm_ref.at[b, pl.ds(nb * bn, bn), :], xblk_ref, sem)
                cp.start()
                cp.wait()
                z0b = jax.lax.dot_general(
                    xblk_ref[...], w0_ref[...], _DN,
                    preferred_element_type=jnp.float32)
                y_ref[...] += jnp.dot(mask_block(nb) * scale, z0b,
                                      preferred_element_type=jnp.float32)


def kernel(x, input_ids, W0, b0, W1, b1):
    B, S, D = x.shape
    bm, bn = 512, 256
    mpb = S // bm
    nblks = S // bn
    R = B * S

    ids = input_ids.astype(jnp.int32)
    key = ids[:, :-1] * 1000 + ids[:, 1:]
    keyM = jnp.concatenate(
        [jnp.full((B, 1), -1, jnp.int32), key], axis=1)
    keyN = jnp.concatenate(
        [jnp.full((B, 2), -2, jnp.int32), key[:, :-1]], axis=1)
    bias = (b0 + b1).reshape(1, D)

    y = pl.pallas_call(
        functools.partial(_body, bm=bm, bn=bn, nblks=nblks, mpb=mpb),
        grid=(R // bm,),
        in_specs=[
            pl.BlockSpec((bm, 1), lambda r: (r, 0)),
            pl.BlockSpec((1, 1, S), lambda r: (r // mpb, 0, 0)),
            pl.BlockSpec((bm, D), lambda r: (r, 0)),
            pl.BlockSpec((D, D), lambda r: (0, 0)),
            pl.BlockSpec((D, D), lambda r: (0, 0)),
            pl.BlockSpec((1, D), lambda r: (0, 0)),
            pl.BlockSpec(memory_space=pltpu.MemorySpace.HBM),
        ],
        out_specs=pl.BlockSpec((bm, D), lambda r: (r, 0)),
        out_shape=jax.ShapeDtypeStruct((R, D), jnp.float32),
        scratch_shapes=[
            pltpu.VMEM((bn, D), jnp.float32),
            pltpu.SemaphoreType.DMA,
        ],
        compiler_params=pltpu.CompilerParams(
            dimension_semantics=("arbitrary",),
            vmem_limit_bytes=62 * 1024 * 1024),
    )(keyM.reshape(R, 1), keyN[:, None, :], x.reshape(R, D), W0, W1, bias, x)
    return y.reshape(B, S, D)
